# baseline (speedup 1.0000x reference)
import functools

import jax
import jax.numpy as jnp
from jax import lax
from jax.experimental import pallas as pl
from jax.experimental.pallas import tpu as pltpu
from jax.experimental.pallas import tpu_sc as plsc

EMB = 1024
HEADS = 16
P = 64
BS = 4
N = 4096
NC = 16
R = BS * NC
TOTAL_ROWS = R * P
DH = EMB // HEADS


def _idx_w_body(q_ref, wpts_ref, bpts_ref, ww_ref, bw_ref, gidx_ref, w_ref):
    q = q_ref[...]
    logits_pts = jnp.dot(q, wpts_ref[...], preferred_element_type=jnp.float32)
    logits_pts = logits_pts + bpts_ref[...]
    idx = jnp.floor(jax.nn.sigmoid(logits_pts) * N).astype(jnp.int32)
    idx = jnp.clip(idx, 0, N - 1)
    row_base = (lax.broadcasted_iota(jnp.int32, (R, P), 0) // NC) * N
    gidx_ref[...] = idx + row_base

    logits_w = jnp.dot(q, ww_ref[...], preferred_element_type=jnp.float32)
    logits_w = logits_w + bw_ref[...]
    m = jnp.max(logits_w, axis=-1, keepdims=True)
    e = jnp.exp(logits_w - m)
    w_ref[...] = e / jnp.sum(e, axis=-1, keepdims=True)


def _idx_w(q, W_pts, b_pts, W_w, b_w):
    return pl.pallas_call(
        _idx_w_body,
        out_shape=(
            jax.ShapeDtypeStruct((R, P), jnp.int32),
            jax.ShapeDtypeStruct((R, P), jnp.float32),
        ),
    )(q, W_pts, b_pts.reshape(1, P), W_w, b_w.reshape(1, P))


_NUM_CORES = 2
_NUM_SUBCORES = 16
_NWORK = _NUM_CORES * _NUM_SUBCORES
_ROWS_PER_W = TOTAL_ROWS // _NWORK
_CHUNK = 64


def _gather_body(table_hbm, idx_hbm, out_hbm, idx_v, rows_v, sem):
    wid = lax.axis_index("s") * _NUM_CORES + lax.axis_index("c")
    base = wid * _ROWS_PER_W
    pltpu.sync_copy(idx_hbm.at[pl.ds(base, _ROWS_PER_W)], idx_v)
    for c in range(_ROWS_PER_W // _CHUNK):
        pltpu.async_copy(
            table_hbm.at[idx_v.at[pl.ds(c * _CHUNK, _CHUNK)]], rows_v, sem
        ).wait()
        pltpu.sync_copy(rows_v, out_hbm.at[pl.ds(base + c * _CHUNK, _CHUNK)])


@functools.cache
def _make_gather():
    return pl.kernel(
        _gather_body,
        out_type=jax.ShapeDtypeStruct((TOTAL_ROWS, EMB), jnp.float32),
        mesh=plsc.VectorSubcoreMesh(
            core_axis_name="c", subcore_axis_name="s",
            num_cores=_NUM_CORES, num_subcores=_NUM_SUBCORES),
        scratch_types=[
            pltpu.VMEM((_ROWS_PER_W,), jnp.int32),
            pltpu.VMEM((_CHUNK, EMB), jnp.float32),
            pltpu.SemaphoreType.DMA,
        ],
    )


_RBLK = 8
_GBLK = _RBLK * P
_GRID = R // _RBLK


def _dense_body(g_ref, w_ref, q_ref, wk_ref, bk_ref, wv_ref, bv_ref,
                wp_ref, bp_ref, out_ref):
    f32 = jnp.float32
    bf16 = jnp.bfloat16
    x = (g_ref[...] * w_ref[...]).astype(bf16)
    k = jnp.dot(x, wk_ref[...], preferred_element_type=f32) + bk_ref[...]
    v = jnp.dot(x, wv_ref[...], preferred_element_type=f32) + bv_ref[...]

    jj = lax.broadcasted_iota(jnp.int32, (EMB, HEADS), 0) // DH
    hh = lax.broadcasted_iota(jnp.int32, (EMB, HEADS), 1)
    m = (jj == hh).astype(bf16)
    mt = m.T
    gg = lax.broadcasted_iota(jnp.int32, (_GBLK, _RBLK), 0) // P
    rr = lax.broadcasted_iota(jnp.int32, (_GBLK, _RBLK), 1)
    rep = (gg == rr).astype(bf16)
    rep_t = rep.T

    scaling = float(EMB) ** 0.5
    xq = jnp.dot(rep, q_ref[...].astype(bf16), preferred_element_type=f32)
    kq = (k * xq).astype(bf16)
    e_gh = jnp.dot(kq, m, preferred_element_type=f32) / scaling
    ex = jnp.exp(e_gh)
    s = jnp.dot(rep_t.astype(f32), ex, preferred_element_type=f32)
    sb = jnp.dot(rep.astype(f32), 1.0 / s, preferred_element_type=f32)
    att = (ex * sb).astype(bf16)
    a2 = jnp.dot(att, mt, preferred_element_type=f32)
    va = (v * a2).astype(bf16)
    o = jnp.dot(rep_t, va, preferred_element_type=f32)
    out_ref[...] = (
        jnp.dot(o.astype(bf16), wp_ref[...], preferred_element_type=f32)
        + bp_ref[...]
    )


def _dense(g, w_col, q, W_k, b_k, W_v, b_v, W_p, b_p):
    full = lambda shape: pl.BlockSpec(shape, lambda i: (0, 0))
    return pl.pallas_call(
        _dense_body,
        grid=(_GRID,),
        in_specs=[
            pl.BlockSpec((_GBLK, EMB), lambda i: (i, 0)),
            pl.BlockSpec((_GBLK, 1), lambda i: (i, 0)),
            pl.BlockSpec((_RBLK, EMB), lambda i: (i, 0)),
            full((EMB, EMB)),
            full((1, EMB)),
            full((EMB, EMB)),
            full((1, EMB)),
            full((EMB, EMB)),
            full((1, EMB)),
        ],
        out_specs=pl.BlockSpec((_RBLK, EMB), lambda i: (i, 0)),
        out_shape=jax.ShapeDtypeStruct((R, EMB), jnp.float32),
    )(g, w_col, q, W_k.astype(jnp.bfloat16), b_k.reshape(1, EMB),
      W_v.astype(jnp.bfloat16), b_v.reshape(1, EMB),
      W_p.astype(jnp.bfloat16), b_p.reshape(1, EMB))


def kernel(input, query, W_pts, b_pts, W_w, b_w, W_k, b_k, W_v, b_v, W_p, b_p):
    q = query.reshape(R, EMB)
    gidx, w = _idx_w(q, W_pts, b_pts, W_w, b_w)
    table = input.reshape(BS * N, EMB)
    g = _make_gather()(table, gidx.reshape(TOTAL_ROWS))
    out = _dense(g, w.reshape(TOTAL_ROWS, 1), q, W_k, b_k, W_v, b_v, W_p, b_p)
    return out.reshape(BS, NC, EMB)


# speedup vs baseline: 1.0509x; 1.0509x over previous
import functools

import jax
import jax.numpy as jnp
from jax import lax
from jax.experimental import pallas as pl
from jax.experimental.pallas import tpu as pltpu
from jax.experimental.pallas import tpu_sc as plsc

EMB = 1024
HEADS = 16
P = 64
BS = 4
N = 4096
NC = 16
R = BS * NC
TOTAL_ROWS = R * P
DH = EMB // HEADS


def _idx_w_body(q_ref, wpts_ref, bpts_ref, ww_ref, bw_ref, gidx_ref, w_ref):
    q = q_ref[...]
    logits_pts = jnp.dot(q, wpts_ref[...], preferred_element_type=jnp.float32)
    logits_pts = logits_pts + bpts_ref[...]
    idx = jnp.floor(jax.nn.sigmoid(logits_pts) * N).astype(jnp.int32)
    idx = jnp.clip(idx, 0, N - 1)
    row_base = (lax.broadcasted_iota(jnp.int32, (R, P), 0) // NC) * N
    gidx_ref[...] = idx + row_base

    logits_w = jnp.dot(q, ww_ref[...], preferred_element_type=jnp.float32)
    logits_w = logits_w + bw_ref[...]
    m = jnp.max(logits_w, axis=-1, keepdims=True)
    e = jnp.exp(logits_w - m)
    w_ref[...] = e / jnp.sum(e, axis=-1, keepdims=True)


def _idx_w(q, W_pts, b_pts, W_w, b_w):
    return pl.pallas_call(
        _idx_w_body,
        out_shape=(
            jax.ShapeDtypeStruct((R, P), jnp.int32),
            jax.ShapeDtypeStruct((R, P), jnp.float32),
        ),
    )(q, W_pts, b_pts.reshape(1, P), W_w, b_w.reshape(1, P))


_NUM_CORES = 2
_NUM_SUBCORES = 16
_NWORK = _NUM_CORES * _NUM_SUBCORES
_ROWS_PER_W = TOTAL_ROWS // _NWORK
_CHUNK = 64


def _gather_body(table_hbm, idx_hbm, out_hbm, idx_v, rows_v, sem):
    wid = lax.axis_index("s") * _NUM_CORES + lax.axis_index("c")
    base = wid * _ROWS_PER_W
    pltpu.sync_copy(idx_hbm.at[pl.ds(base, _ROWS_PER_W)], idx_v)
    for c in range(_ROWS_PER_W // _CHUNK):
        pltpu.async_copy(
            table_hbm.at[idx_v.at[pl.ds(c * _CHUNK, _CHUNK)]], rows_v, sem
        ).wait()
        pltpu.sync_copy(rows_v, out_hbm.at[pl.ds(base + c * _CHUNK, _CHUNK)])


@functools.cache
def _make_gather():
    return pl.kernel(
        _gather_body,
        out_type=jax.ShapeDtypeStruct((TOTAL_ROWS, EMB), jnp.float32),
        mesh=plsc.VectorSubcoreMesh(
            core_axis_name="c", subcore_axis_name="s",
            num_cores=_NUM_CORES, num_subcores=_NUM_SUBCORES),
        scratch_types=[
            pltpu.VMEM((_ROWS_PER_W,), jnp.int32),
            pltpu.VMEM((_CHUNK, EMB), jnp.float32),
            pltpu.SemaphoreType.DMA,
        ],
    )


_RBLK = 16
_GBLK = _RBLK * P
_GRID = R // _RBLK


def _dense_body(g_ref, w_ref, q_ref, wk_ref, bk_ref, wv_ref, bv_ref,
                wp_ref, bp_ref, out_ref):
    f32 = jnp.float32
    bf16 = jnp.bfloat16
    x = (g_ref[...] * w_ref[...]).astype(bf16)
    k = (jnp.dot(x, wk_ref[...], preferred_element_type=f32)
         .astype(bf16) + bk_ref[...])
    v = (jnp.dot(x, wv_ref[...], preferred_element_type=f32)
         .astype(bf16) + bv_ref[...])

    jj = lax.broadcasted_iota(jnp.int32, (EMB, HEADS), 0) // DH
    hh = lax.broadcasted_iota(jnp.int32, (EMB, HEADS), 1)
    ind = (jj == hh)
    inv_scale = float(EMB) ** -0.5
    m_s = jnp.where(ind, inv_scale, 0.0).astype(bf16)
    mt = ind.T.astype(bf16)
    mt_f = ind.T.astype(f32)
    gg = lax.broadcasted_iota(jnp.int32, (_GBLK, _RBLK), 0) // P
    rr = lax.broadcasted_iota(jnp.int32, (_GBLK, _RBLK), 1)
    rep = (gg == rr).astype(bf16)
    rep_t = rep.T

    xq = jnp.dot(rep, q_ref[...].astype(bf16),
                 preferred_element_type=f32).astype(bf16)
    e_gh = jnp.dot(k * xq, m_s, preferred_element_type=f32)
    ex = jnp.exp(e_gh)
    a2 = jnp.dot(ex.astype(bf16), mt,
                 preferred_element_type=f32).astype(bf16)
    o_un = jnp.dot(rep_t, v * a2, preferred_element_type=f32)
    s = jnp.dot(rep_t.astype(f32), ex, preferred_element_type=f32)
    norm = jnp.dot(1.0 / s, mt_f, preferred_element_type=f32)
    o = o_un * norm
    out_ref[...] = (
        jnp.dot(o.astype(bf16), wp_ref[...], preferred_element_type=f32)
        + bp_ref[...]
    )


def _dense(g, w_col, q, W_k, b_k, W_v, b_v, W_p, b_p):
    full = lambda shape: pl.BlockSpec(shape, lambda i: (0, 0))
    return pl.pallas_call(
        _dense_body,
        grid=(_GRID,),
        in_specs=[
            pl.BlockSpec((_GBLK, EMB), lambda i: (i, 0)),
            pl.BlockSpec((_GBLK, 1), lambda i: (i, 0)),
            pl.BlockSpec((_RBLK, EMB), lambda i: (i, 0)),
            full((EMB, EMB)),
            full((1, EMB)),
            full((EMB, EMB)),
            full((1, EMB)),
            full((EMB, EMB)),
            full((1, EMB)),
        ],
        out_specs=pl.BlockSpec((_RBLK, EMB), lambda i: (i, 0)),
        out_shape=jax.ShapeDtypeStruct((R, EMB), jnp.float32),
    )(g, w_col, q, W_k.astype(jnp.bfloat16),
      b_k.reshape(1, EMB).astype(jnp.bfloat16),
      W_v.astype(jnp.bfloat16), b_v.reshape(1, EMB).astype(jnp.bfloat16),
      W_p.astype(jnp.bfloat16), b_p.reshape(1, EMB))


def kernel(input, query, W_pts, b_pts, W_w, b_w, W_k, b_k, W_v, b_v, W_p, b_p):
    q = query.reshape(R, EMB)
    gidx, w = _idx_w(q, W_pts, b_pts, W_w, b_w)
    table = input.reshape(BS * N, EMB)
    g = _make_gather()(table, gidx.reshape(TOTAL_ROWS))
    out = _dense(g, w.reshape(TOTAL_ROWS, 1), q, W_k, b_k, W_v, b_v, W_p, b_p)
    return out.reshape(BS, NC, EMB)
